# baseline (speedup 1.0000x reference)
import jax
import jax.numpy as jnp
from jax import lax
from jax.experimental import pallas as pl
from jax.experimental.pallas import tpu as pltpu
from jax.experimental.pallas import tpu_sc as plsc

D = 64
SEQ = 200
LANES = 16
NC, NS = 2, 16
NW = NC * NS

CROWS = SEQ
NBUF = 4
RUNROLL = 8


def _enc_body(ids_hbm, table_hbm, pos_hbm, out_hbm,
              idx_v, rows_v, pos_v, gsem, osem):
    n = out_hbm.shape[0]
    rpw = n // NW
    nch = rpw // CROWS

    wid = lax.axis_index("s") * NC + lax.axis_index("c")
    base_w = wid * rpw

    pltpu.sync_copy(pos_hbm, pos_v)
    pltpu.sync_copy(ids_hbm.at[wid], idx_v)

    def prefill(b):
        def pf_body(r8, carry):
            for k in range(RUNROLL):
                r = r8 * RUNROLL + k
                for c in range(D // LANES):
                    sl = pl.ds(c * LANES, LANES)
                    rows_v[b, r, sl] = pos_v[r, sl]
            return carry
        lax.fori_loop(0, CROWS // RUNROLL, pf_body, 0)

    def group_body(g, carry):
        for b in range(NBUF):
            c = g * NBUF + b

            @pl.when(jnp.logical_and(c >= NBUF, c < nch))
            def _drain_out():
                pltpu.make_async_copy(
                    rows_v.at[b],
                    out_hbm.at[pl.ds(base_w + (c - NBUF) * CROWS, CROWS)],
                    osem.at[b],
                ).wait()

            @pl.when(c < nch)
            def _issue():
                prefill(b)
                pltpu.async_copy(table_hbm.at[idx_v.at[c]], rows_v.at[b],
                                 gsem.at[b], add=True)

            d = c - (NBUF - 1)
            b2 = (b + 1) % NBUF

            @pl.when(jnp.logical_and(d >= 0, d < nch))
            def _complete():
                pltpu.make_async_copy(table_hbm.at[idx_v.at[d]],
                                      rows_v.at[b2], gsem.at[b2]).wait()
                pltpu.async_copy(
                    rows_v.at[b2],
                    out_hbm.at[pl.ds(base_w + d * CROWS, CROWS)],
                    osem.at[b2],
                )
        return carry

    lax.fori_loop(0, nch // NBUF + 1, group_body, 0)

    for b in range(NBUF):
        c_last = nch - NBUF + b
        pltpu.make_async_copy(
            rows_v.at[b],
            out_hbm.at[pl.ds(base_w + c_last * CROWS, CROWS)],
            osem.at[b],
        ).wait()


def kernel(input_ids, embedding, positional_encoding):
    b, s = input_ids.shape
    n = b * s
    rpw = n // NW
    nch = rpw // CROWS
    ids_r = input_ids.reshape(NW, nch, CROWS).astype(jnp.int32)
    mesh = plsc.VectorSubcoreMesh(core_axis_name="c", subcore_axis_name="s")
    out = pl.kernel(
        _enc_body,
        out_type=jax.ShapeDtypeStruct((n, D), jnp.float32),
        mesh=mesh,
        compiler_params=pltpu.CompilerParams(use_tc_tiling_on_sc=False),
        scratch_types=[
            pltpu.VMEM((nch, CROWS), jnp.int32),
            pltpu.VMEM((NBUF, CROWS, D), jnp.float32),
            pltpu.VMEM((SEQ, D), jnp.float32),
            pltpu.SemaphoreType.DMA((NBUF,)),
            pltpu.SemaphoreType.DMA((NBUF,)),
        ],
    )(ids_r, embedding, positional_encoding)
    return out.reshape(b, s, D)
